# baseline (speedup 1.0000x reference)
import functools

import jax
import jax.numpy as jnp
from jax import lax
from jax.experimental import pallas as pl
from jax.experimental.pallas import tpu as pltpu
from jax.experimental.pallas import tpu_sc as plsc

_H = 10
_HP = 16
_NC = 2
_NS = 16
_NT = 3
_CHUNK = 8192
_UNROLL = 8


@functools.cache
def _sc_segment_sum(n_pad: int, e_pad: int):
  ept = e_pad // _NT
  nch = ept // _CHUNK
  mesh = plsc.VectorSubcoreMesh(core_axis_name="c", subcore_axis_name="s")

  @functools.partial(
      pl.kernel,
      out_type=jax.ShapeDtypeStruct((_NT, _HP, n_pad), jnp.float32),
      mesh=mesh,
      compiler_params=pltpu.CompilerParams(
          use_tc_tiling_on_sc=False, needs_layout_passes=False),
      scratch_types=[
          pltpu.VMEM((n_pad,), jnp.float32),
          pltpu.VMEM((n_pad,), jnp.float32),
          pltpu.VMEM((n_pad,), jnp.float32),
          pltpu.VMEM((2, _CHUNK), jnp.int32),
          pltpu.VMEM((2, _CHUNK), jnp.int32),
          pltpu.VMEM((2, _CHUNK), jnp.float32),
          pltpu.SemaphoreType.DMA,
          pltpu.SemaphoreType.DMA,
      ],
  )
  def sc_kernel(y_hbm, src_hbm, dst_hbm, ew_hbm, out_hbm,
                ycol, acc, acc2, sring, dring, wring, sem0, sem1):
    c = lax.axis_index("c")
    s = lax.axis_index("s")
    wid = s * _NC + c
    sems = (sem0, sem1)

    @pl.when(wid < _H * _NT)
    def _():
      f = wid % _H
      t = wid // _H
      base = t * ept

      pltpu.sync_copy(y_hbm.at[f], ycol)

      @pl.loop(0, n_pad // _HP)
      def _(i):
        acc[pl.ds(i * _HP, _HP)] = jnp.zeros((_HP,), jnp.float32)
        acc2[pl.ds(i * _HP, _HP)] = jnp.zeros((_HP,), jnp.float32)

      def fire(k, b):
        off = base + k * _CHUNK
        pltpu.async_copy(src_hbm.at[pl.ds(off, _CHUNK)], sring.at[b], sems[b])
        pltpu.async_copy(dst_hbm.at[pl.ds(off, _CHUNK)], dring.at[b], sems[b])
        pltpu.async_copy(ew_hbm.at[pl.ds(off, _CHUNK)], wring.at[b], sems[b])

      def drain(k, b):
        off = base + k * _CHUNK
        pltpu.make_async_copy(
            src_hbm.at[pl.ds(off, _CHUNK)], sring.at[b], sems[b]).wait()
        pltpu.make_async_copy(
            dst_hbm.at[pl.ds(off, _CHUNK)], dring.at[b], sems[b]).wait()
        pltpu.make_async_copy(
            ew_hbm.at[pl.ds(off, _CHUNK)], wring.at[b], sems[b]).wait()

      def process(b):
        @pl.loop(0, _CHUNK // (16 * _UNROLL))
        def _(g):
          o0 = g * (16 * _UNROLL)
          for u in range(_UNROLL):
            o = pl.multiple_of(o0 + u * 16, 16)
            s16 = sring[b, pl.ds(o, 16)]
            d16 = dring[b, pl.ds(o, 16)]
            w16 = wring[b, pl.ds(o, 16)]
            v = plsc.load_gather(ycol, [s16])
            plsc.addupdate_scatter(acc if u % 2 == 0 else acc2,
                                   [d16], v * w16)

      fire(0, 0)

      @pl.loop(0, nch, step=2)
      def _(k):
        fire(k + 1, 1)
        drain(k, 0)
        process(0)

        @pl.when(k + 2 < nch)
        def _():
          fire(k + 2, 0)

        drain(k + 1, 1)
        process(1)

      @pl.loop(0, n_pad // _HP)
      def _(i):
        o = pl.ds(i * _HP, _HP)
        acc[o] = acc[o] + acc2[o]

      pltpu.sync_copy(acc, out_hbm.at[t, f])

  return sc_kernel


def _tc_pre(x0, wz, wr, bias):
  n, d = x0.shape
  rb = n

  def body(x_ref, wz_ref, wr_ref, b_ref, z_ref, r_ref):
    x = x_ref[...]
    z_ref[...] = jnp.dot(x, wz_ref[...], preferred_element_type=jnp.float32)
    r_ref[...] = (jnp.dot(x, wr_ref[...], preferred_element_type=jnp.float32)
                  + b_ref[...])

  return pl.pallas_call(
      body,
      grid=(n // rb,),
      in_specs=[
          pl.BlockSpec((rb, d), lambda i: (i, 0)),
          pl.BlockSpec((d, _HP), lambda i: (0, 0)),
          pl.BlockSpec((d, _HP), lambda i: (0, 0)),
          pl.BlockSpec((1, _HP), lambda i: (0, 0)),
      ],
      out_specs=[
          pl.BlockSpec((rb, _HP), lambda i: (i, 0)),
          pl.BlockSpec((rb, _HP), lambda i: (i, 0)),
      ],
      out_shape=[jax.ShapeDtypeStruct((n, _HP), jnp.float32)] * 2,
  )(x0, wz, wr, bias)


def _row_mask(rb):
  return lax.broadcasted_iota(jnp.int32, (_HP, rb), 0) < _H


def _tc_mid(parts, r_prev, wz, wr, bias):
  n = r_prev.shape[1]
  rb = n

  def body(p_ref, rp_ref, wz_ref, wr_ref, b_ref, z_ref, r_ref):
    x = jnp.maximum(p_ref[0] + p_ref[1] + p_ref[2] + rp_ref[...], 0.0)
    x = jnp.where(_row_mask(rb), x, 0.0)
    z_ref[...] = jnp.dot(wz_ref[...], x, preferred_element_type=jnp.float32)
    r_ref[...] = (jnp.dot(wr_ref[...], x, preferred_element_type=jnp.float32)
                  + b_ref[...])

  return pl.pallas_call(
      body,
      grid=(n // rb,),
      in_specs=[
          pl.BlockSpec((_NT, _HP, rb), lambda i: (0, 0, i)),
          pl.BlockSpec((_HP, rb), lambda i: (0, i)),
          pl.BlockSpec((_HP, _HP), lambda i: (0, 0)),
          pl.BlockSpec((_HP, _HP), lambda i: (0, 0)),
          pl.BlockSpec((_HP, 1), lambda i: (0, 0)),
      ],
      out_specs=[
          pl.BlockSpec((_HP, rb), lambda i: (0, i)),
          pl.BlockSpec((_HP, rb), lambda i: (0, i)),
      ],
      out_shape=[jax.ShapeDtypeStruct((_HP, n), jnp.float32)] * 2,
  )(parts, r_prev, wz, wr, bias)


def _tc_post(parts, r_prev, wlin, blin):
  n = r_prev.shape[1]
  rb = n

  def body(p_ref, rp_ref, wl_ref, bl_ref, o_ref):
    x = jnp.maximum(p_ref[0] + p_ref[1] + p_ref[2] + rp_ref[...], 0.0)
    x = jnp.where(_row_mask(rb), x, 0.0)
    o_ref[...] = (jnp.sum(x * wl_ref[...], axis=0, keepdims=True)
                  + bl_ref[...])

  return pl.pallas_call(
      body,
      grid=(n // rb,),
      in_specs=[
          pl.BlockSpec((_NT, _HP, rb), lambda i: (0, 0, i)),
          pl.BlockSpec((_HP, rb), lambda i: (0, i)),
          pl.BlockSpec((_HP, 1), lambda i: (0, 0)),
          pl.BlockSpec((1, 1), lambda i: (0, 0)),
      ],
      out_specs=pl.BlockSpec((1, rb), lambda i: (0, i)),
      out_shape=jax.ShapeDtypeStruct((1, n), jnp.float32),
  )(parts, r_prev, wlin, blin)


def _pad_w(w):
  h, d = w.shape
  return jnp.zeros((_HP, max(d, _HP)), jnp.float32).at[:h, :d].set(w)


def _pad_col(b):
  return jnp.zeros((_HP, 1), jnp.float32).at[: b.shape[0], 0].set(b)


def kernel(x0, edge_index, edge_weights, W1_rel, b1, W1_root,
           W2_rel, b2, W2_root, W_lin, b_lin):
  n, _ = x0.shape
  e = edge_index.shape[1]
  src = edge_index[0]
  dst = edge_index[1]

  n_pad = -(-n // 128) * 128
  x0p = jnp.zeros((n_pad, x0.shape[1]), jnp.float32).at[:n].set(x0)

  granule = _NT * 2 * _CHUNK
  e_pad = -(-e // granule) * granule
  pad = e_pad - e
  if pad:
    src = jnp.concatenate([src, jnp.zeros((pad,), src.dtype)])
    dst = jnp.concatenate([dst, jnp.zeros((pad,), dst.dtype)])
    edge_weights = jnp.concatenate(
        [edge_weights, jnp.zeros((pad,), edge_weights.dtype)])

  wz1, wr1 = _pad_w(W1_rel).T, _pad_w(W1_root).T
  b1p = _pad_col(b1).T
  wz2, wr2, b2p = _pad_w(W2_rel), _pad_w(W2_root), _pad_col(b2)
  wlp = _pad_col(W_lin[0])
  blp = b_lin.reshape(1, 1)

  sc_seg = _sc_segment_sum(n_pad, e_pad)

  z0, r0 = _tc_pre(x0p, wz1, wr1, b1p)
  z0t, r0t = z0.T, r0.T
  p1 = sc_seg(z0t, src, dst, edge_weights)
  z1, r1 = _tc_mid(p1, r0t, wz2, wr2, b2p)
  p2 = sc_seg(z1, src, dst, edge_weights)
  out = _tc_post(p2, r1, wlp, blp)
  return out[0, :n, None]


# speedup vs baseline: 1.5024x; 1.5024x over previous
import functools

import jax
import jax.numpy as jnp
from jax import lax
from jax.experimental import pallas as pl
from jax.experimental.pallas import tpu as pltpu
from jax.experimental.pallas import tpu_sc as plsc

_H = 10
_HP = 16
_NC = 2
_NS = 16
_NT = 3
_CHUNK = 8192
_UNROLL = 8


@functools.cache
def _sc_segment_sum(n_pad: int, e_pad: int):
  ept = e_pad // _NT
  nch = ept // _CHUNK
  mesh = plsc.VectorSubcoreMesh(core_axis_name="c", subcore_axis_name="s")

  @functools.partial(
      pl.kernel,
      out_type=jax.ShapeDtypeStruct((_NT, _HP, n_pad), jnp.float32),
      mesh=mesh,
      compiler_params=pltpu.CompilerParams(
          use_tc_tiling_on_sc=False, needs_layout_passes=False),
      scratch_types=[
          pltpu.VMEM((n_pad,), jnp.float32),
          pltpu.VMEM((n_pad,), jnp.float32),
          pltpu.VMEM((2, _CHUNK), jnp.int32),
          pltpu.VMEM((2, _CHUNK), jnp.int32),
          pltpu.VMEM((2, _CHUNK), jnp.float32),
          pltpu.SemaphoreType.DMA,
          pltpu.SemaphoreType.DMA,
      ],
  )
  def sc_kernel(y_hbm, src_hbm, dst_hbm, ew_hbm, out_hbm,
                ycol, acc, sring, dring, wring, sem0, sem1):
    c = lax.axis_index("c")
    s = lax.axis_index("s")
    wid = s * _NC + c
    sems = (sem0, sem1)

    @pl.when(wid < _H * _NT)
    def _():
      f = wid % _H
      t = wid // _H
      base = t * ept

      pltpu.sync_copy(y_hbm.at[f], ycol)

      @pl.loop(0, n_pad // _HP)
      def _(i):
        acc[pl.ds(i * _HP, _HP)] = jnp.zeros((_HP,), jnp.float32)

      def fire(k, b):
        off = base + k * _CHUNK
        pltpu.async_copy(src_hbm.at[pl.ds(off, _CHUNK)], sring.at[b], sems[b])
        pltpu.async_copy(dst_hbm.at[pl.ds(off, _CHUNK)], dring.at[b], sems[b])
        pltpu.async_copy(ew_hbm.at[pl.ds(off, _CHUNK)], wring.at[b], sems[b])

      def drain(k, b):
        off = base + k * _CHUNK
        pltpu.make_async_copy(
            src_hbm.at[pl.ds(off, _CHUNK)], sring.at[b], sems[b]).wait()
        pltpu.make_async_copy(
            dst_hbm.at[pl.ds(off, _CHUNK)], dring.at[b], sems[b]).wait()
        pltpu.make_async_copy(
            ew_hbm.at[pl.ds(off, _CHUNK)], wring.at[b], sems[b]).wait()

      def process(b):
        @pl.loop(0, _CHUNK // (16 * _UNROLL))
        def _(g):
          o0 = g * (16 * _UNROLL)
          offs = [pl.ds(pl.multiple_of(o0 + u * 16, 16), 16)
                  for u in range(_UNROLL)]
          ss = [sring[b, o] for o in offs]
          dd = [dring[b, o] for o in offs]
          ww = [wring[b, o] for o in offs]
          vv = [plsc.load_gather(ycol, [s16]) for s16 in ss]
          pp = [v * w for v, w in zip(vv, ww)]
          for d16, p in zip(dd, pp):
            plsc.addupdate_scatter(acc, [d16], p)

      fire(0, 0)

      @pl.loop(0, nch, step=2)
      def _(k):
        fire(k + 1, 1)
        drain(k, 0)
        process(0)

        @pl.when(k + 2 < nch)
        def _():
          fire(k + 2, 0)

        drain(k + 1, 1)
        process(1)

      pltpu.sync_copy(acc, out_hbm.at[t, f])

  return sc_kernel


def _tc_pre(x0, wz, wr, bias):
  n, d = x0.shape
  rb = n

  def body(x_ref, wz_ref, wr_ref, b_ref, z_ref, r_ref):
    x = x_ref[...]
    z_ref[...] = jnp.dot(x, wz_ref[...], preferred_element_type=jnp.float32)
    r_ref[...] = (jnp.dot(x, wr_ref[...], preferred_element_type=jnp.float32)
                  + b_ref[...])

  return pl.pallas_call(
      body,
      grid=(n // rb,),
      in_specs=[
          pl.BlockSpec((rb, d), lambda i: (i, 0)),
          pl.BlockSpec((d, _HP), lambda i: (0, 0)),
          pl.BlockSpec((d, _HP), lambda i: (0, 0)),
          pl.BlockSpec((1, _HP), lambda i: (0, 0)),
      ],
      out_specs=[
          pl.BlockSpec((rb, _HP), lambda i: (i, 0)),
          pl.BlockSpec((rb, _HP), lambda i: (i, 0)),
      ],
      out_shape=[jax.ShapeDtypeStruct((n, _HP), jnp.float32)] * 2,
  )(x0, wz, wr, bias)


def _row_mask(rb):
  return lax.broadcasted_iota(jnp.int32, (_HP, rb), 0) < _H


def _tc_mid(parts, r_prev, wz, wr, bias):
  n = r_prev.shape[1]
  rb = n

  def body(p_ref, rp_ref, wz_ref, wr_ref, b_ref, z_ref, r_ref):
    x = jnp.maximum(p_ref[0] + p_ref[1] + p_ref[2] + rp_ref[...], 0.0)
    x = jnp.where(_row_mask(rb), x, 0.0)
    z_ref[...] = jnp.dot(wz_ref[...], x, preferred_element_type=jnp.float32)
    r_ref[...] = (jnp.dot(wr_ref[...], x, preferred_element_type=jnp.float32)
                  + b_ref[...])

  return pl.pallas_call(
      body,
      grid=(n // rb,),
      in_specs=[
          pl.BlockSpec((_NT, _HP, rb), lambda i: (0, 0, i)),
          pl.BlockSpec((_HP, rb), lambda i: (0, i)),
          pl.BlockSpec((_HP, _HP), lambda i: (0, 0)),
          pl.BlockSpec((_HP, _HP), lambda i: (0, 0)),
          pl.BlockSpec((_HP, 1), lambda i: (0, 0)),
      ],
      out_specs=[
          pl.BlockSpec((_HP, rb), lambda i: (0, i)),
          pl.BlockSpec((_HP, rb), lambda i: (0, i)),
      ],
      out_shape=[jax.ShapeDtypeStruct((_HP, n), jnp.float32)] * 2,
  )(parts, r_prev, wz, wr, bias)


def _tc_post(parts, r_prev, wlin, blin):
  n = r_prev.shape[1]
  rb = n

  def body(p_ref, rp_ref, wl_ref, bl_ref, o_ref):
    x = jnp.maximum(p_ref[0] + p_ref[1] + p_ref[2] + rp_ref[...], 0.0)
    x = jnp.where(_row_mask(rb), x, 0.0)
    o_ref[...] = (jnp.sum(x * wl_ref[...], axis=0, keepdims=True)
                  + bl_ref[...])

  return pl.pallas_call(
      body,
      grid=(n // rb,),
      in_specs=[
          pl.BlockSpec((_NT, _HP, rb), lambda i: (0, 0, i)),
          pl.BlockSpec((_HP, rb), lambda i: (0, i)),
          pl.BlockSpec((_HP, 1), lambda i: (0, 0)),
          pl.BlockSpec((1, 1), lambda i: (0, 0)),
      ],
      out_specs=pl.BlockSpec((1, rb), lambda i: (0, i)),
      out_shape=jax.ShapeDtypeStruct((1, n), jnp.float32),
  )(parts, r_prev, wlin, blin)


def _pad_w(w):
  h, d = w.shape
  return jnp.zeros((_HP, max(d, _HP)), jnp.float32).at[:h, :d].set(w)


def _pad_col(b):
  return jnp.zeros((_HP, 1), jnp.float32).at[: b.shape[0], 0].set(b)


def kernel(x0, edge_index, edge_weights, W1_rel, b1, W1_root,
           W2_rel, b2, W2_root, W_lin, b_lin):
  n, _ = x0.shape
  e = edge_index.shape[1]
  src = edge_index[0]
  dst = edge_index[1]

  n_pad = -(-n // 128) * 128
  x0p = jnp.zeros((n_pad, x0.shape[1]), jnp.float32).at[:n].set(x0)

  granule = _NT * 2 * _CHUNK
  e_pad = -(-e // granule) * granule
  pad = e_pad - e
  if pad:
    src = jnp.concatenate([src, jnp.zeros((pad,), src.dtype)])
    dst = jnp.concatenate([dst, jnp.zeros((pad,), dst.dtype)])
    edge_weights = jnp.concatenate(
        [edge_weights, jnp.zeros((pad,), edge_weights.dtype)])

  wz1, wr1 = _pad_w(W1_rel).T, _pad_w(W1_root).T
  b1p = _pad_col(b1).T
  wz2, wr2, b2p = _pad_w(W2_rel), _pad_w(W2_root), _pad_col(b2)
  wlp = _pad_col(W_lin[0])
  blp = b_lin.reshape(1, 1)

  sc_seg = _sc_segment_sum(n_pad, e_pad)

  z0, r0 = _tc_pre(x0p, wz1, wr1, b1p)
  z0t, r0t = z0.T, r0.T
  p1 = sc_seg(z0t, src, dst, edge_weights)
  z1, r1 = _tc_mid(p1, r0t, wz2, wr2, b2p)
  p2 = sc_seg(z1, src, dst, edge_weights)
  out = _tc_post(p2, r1, wlp, blp)
  return out[0, :n, None]
